# baseline (speedup 1.0000x reference)
import functools

import jax
import jax.numpy as jnp
from jax import lax
from jax.experimental import pallas as pl
from jax.experimental.pallas import tpu as pltpu
from jax.experimental.pallas import tpu_sc as plsc

NC = 2
NS = 16
NW = NC * NS
CHUNK = 100

_f32 = jnp.float32


def _mesh():
    return plsc.VectorSubcoreMesh(
        core_axis_name="c", subcore_axis_name="s", num_cores=NC, num_subcores=NS
    )


def _fill_zeros(ref, n_rows, d):
    zeros16 = jnp.zeros((16,), _f32)

    def body(i, _):
        for k in range(d // 16):
            ref[i, pl.ds(k * 16, 16)] = zeros16
        return 0

    lax.fori_loop(0, n_rows, body, 0)


def _make_seg_sum(n, npad, d, n_chunks_per_tile, k):
    rows_per_tile = npad // NS
    assert n_chunks_per_tile % k == 0
    assert k * CHUNK >= rows_per_tile
    nb = n_chunks_per_tile // k

    @functools.partial(
        pl.kernel,
        out_type=jax.ShapeDtypeStruct((NC * npad, d), _f32),
        mesh=_mesh(),
        compiler_params=pltpu.CompilerParams(use_tc_tiling_on_sc=False),
        scratch_types=[
            pltpu.VMEM((n_chunks_per_tile, CHUNK), jnp.int32),
            pltpu.VMEM((n_chunks_per_tile, CHUNK), jnp.int32),
            pltpu.VMEM((k * CHUNK, d), _f32),
            pltpu.VMEM_SHARED((npad, d), _f32),
            pltpu.SemaphoreType.DMA,
            pltpu.SemaphoreType.DMA,
        ],
    )
    def seg_sum(h_hbm, src_hbm, dst_hbm, out_hbm, sidx, didx, rows, acc, semg, sems):
        cid = lax.axis_index("c")
        sid = lax.axis_index("s")
        wid = cid * NS + sid

        _fill_zeros(rows, rows_per_tile, d)
        pltpu.sync_copy(
            rows.at[pl.ds(0, rows_per_tile)],
            acc.at[pl.ds(sid * rows_per_tile, rows_per_tile)],
        )

        pltpu.sync_copy(src_hbm.at[wid], sidx)
        pltpu.sync_copy(dst_hbm.at[wid], didx)
        plsc.subcore_barrier()

        def batch(b, _):
            j0 = b * k

            def fire_g(j2, _):
                pltpu.async_copy(
                    h_hbm.at[sidx.at[j0 + j2]],
                    rows.at[pl.ds(j2 * CHUNK, CHUNK)],
                    semg,
                )
                return 0

            def drain_g(j2, _):
                pltpu.make_async_copy(
                    h_hbm.at[sidx.at[j0 + j2]],
                    rows.at[pl.ds(j2 * CHUNK, CHUNK)],
                    semg,
                ).wait()
                return 0

            def fire_s(j2, _):
                pltpu.async_copy(
                    rows.at[pl.ds(j2 * CHUNK, CHUNK)],
                    acc.at[didx.at[j0 + j2]],
                    sems,
                    add=True,
                )
                return 0

            def drain_s(j2, _):
                pltpu.make_async_copy(
                    rows.at[pl.ds(j2 * CHUNK, CHUNK)],
                    acc.at[didx.at[j0 + j2]],
                    sems,
                ).wait()
                return 0

            lax.fori_loop(0, k, fire_g, 0)
            lax.fori_loop(0, k, drain_g, 0)
            lax.fori_loop(0, k, fire_s, 0)
            lax.fori_loop(0, k, drain_s, 0)
            return 0

        lax.fori_loop(0, nb, batch, 0)
        plsc.subcore_barrier()

        r0 = sid * rows_per_tile
        pltpu.sync_copy(
            acc.at[pl.ds(r0, rows_per_tile)],
            out_hbm.at[pl.ds(cid * npad + r0, rows_per_tile)],
        )

    return seg_sum


def _make_deg(n, npad, n_chunks_per_tile):
    rows_per_tile = npad // NS

    @functools.partial(
        pl.kernel,
        out_type=jax.ShapeDtypeStruct((NC * npad, 16), _f32),
        mesh=_mesh(),
        compiler_params=pltpu.CompilerParams(use_tc_tiling_on_sc=False),
        scratch_types=[
            pltpu.VMEM((n_chunks_per_tile, CHUNK), jnp.int32),
            pltpu.VMEM((CHUNK, 16), _f32),
            pltpu.VMEM((rows_per_tile, 16), _f32),
            pltpu.VMEM_SHARED((npad, 16), _f32),
            pltpu.SemaphoreType.DMA,
        ],
    )
    def deg_kernel(dst_hbm, out_hbm, didx, ones_v, zb, acc, sem):
        cid = lax.axis_index("c")
        sid = lax.axis_index("s")
        wid = cid * NS + sid

        ones16 = jnp.ones((16,), _f32)

        def fill_ones(i, _):
            ones_v[i, :] = ones16
            return 0

        lax.fori_loop(0, CHUNK, fill_ones, 0)
        _fill_zeros(zb, rows_per_tile, 16)
        pltpu.sync_copy(zb, acc.at[pl.ds(sid * rows_per_tile, rows_per_tile)])

        pltpu.sync_copy(dst_hbm.at[wid], didx)
        plsc.subcore_barrier()

        def fire(j, _):
            pltpu.async_copy(ones_v, acc.at[didx.at[j]], sem, add=True)
            return 0

        def drain(j, _):
            pltpu.make_async_copy(ones_v, acc.at[didx.at[j]], sem).wait()
            return 0

        lax.fori_loop(0, n_chunks_per_tile, fire, 0)
        lax.fori_loop(0, n_chunks_per_tile, drain, 0)
        plsc.subcore_barrier()

        r0 = sid * rows_per_tile
        pltpu.sync_copy(
            acc.at[pl.ds(r0, rows_per_tile)],
            out_hbm.at[pl.ds(cid * npad + r0, rows_per_tile)],
        )

    return deg_kernel


def _dinv_from_parts(p0, p1):
    deg = p0[:, 0:1] + p1[:, 0:1]
    return lax.rsqrt(jnp.maximum(deg, 1.0))


def _mm_body(x_ref, w1_ref, o_ref):
    o_ref[...] = jnp.dot(x_ref[...], w1_ref[...], preferred_element_type=_f32)


def _prescale_body(h_ref, pd_ref, o_ref):
    dinv = _dinv_from_parts(pd_ref[0], pd_ref[1])
    o_ref[...] = h_ref[...] * dinv


def _h2_body(p1_ref, pd_ref, w2_ref, b1_ref, o_ref):
    dinv = _dinv_from_parts(pd_ref[0], pd_ref[1])
    s1 = p1_ref[0] + p1_ref[1]
    hidden = jax.nn.relu(dinv * s1 + b1_ref[...])
    o_ref[...] = jnp.dot(hidden, w2_ref[...], preferred_element_type=_f32) * dinv


def _dec_body(p2_ref, pd_ref, b2_ref, o_ref, z_ref, *, n, bi):
    i = pl.program_id(0)

    @pl.when(i == 0)
    def _():
        dinv = _dinv_from_parts(pd_ref[0, :n], pd_ref[1, :n])
        z_ref[...] = dinv * (p2_ref[0, :n] + p2_ref[1, :n]) + b2_ref[...]

    zi = z_ref[pl.ds(i * bi, bi), :]
    prod = lax.dot_general(
        zi, z_ref[...], (((1,), (1,)), ((), ())),
        preferred_element_type=_f32,
    )
    o_ref[...] = 0.5 * jnp.tanh(0.5 * prod) + 0.5


def kernel(X, edge_index, W1, b1, W2, b2):
    n, d_in = X.shape
    d_h = W1.shape[1]
    d_lat = W2.shape[1]
    e = edge_index.shape[1]

    n_chunks_per_tile = e // (NW * CHUNK)
    ei = edge_index.reshape(2, NW, n_chunks_per_tile, CHUNK)
    src2d, dst2d = ei[0], ei[1]
    npad = ((n + NS * 8 - 1) // (NS * 8)) * NS * 8

    pdeg = _make_deg(n, npad, n_chunks_per_tile)(dst2d).reshape(NC, npad, 16)

    br = 1000
    grid = (n // br,)
    h1 = pl.pallas_call(
        _mm_body,
        grid=grid,
        in_specs=[
            pl.BlockSpec((br, d_in), lambda i: (i, 0)),
            pl.BlockSpec((d_in, d_h), lambda i: (0, 0)),
        ],
        out_specs=pl.BlockSpec((br, d_h), lambda i: (i, 0)),
        out_shape=jax.ShapeDtypeStruct((n, d_h), _f32),
    )(X, W1)

    h1p = pl.pallas_call(
        _prescale_body,
        grid=grid,
        in_specs=[
            pl.BlockSpec((br, d_h), lambda i: (i, 0)),
            pl.BlockSpec((NC, br, 16), lambda i: (0, i, 0)),
        ],
        out_specs=pl.BlockSpec((br, d_h), lambda i: (i, 0)),
        out_shape=jax.ShapeDtypeStruct((n, d_h), _f32),
    )(h1, pdeg)

    p1 = _make_seg_sum(n, npad, d_h, n_chunks_per_tile, 10)(h1p, src2d, dst2d)
    p1 = p1.reshape(NC, npad, d_h)

    h2p = pl.pallas_call(
        _h2_body,
        grid=grid,
        in_specs=[
            pl.BlockSpec((NC, br, d_h), lambda i: (0, i, 0)),
            pl.BlockSpec((NC, br, 16), lambda i: (0, i, 0)),
            pl.BlockSpec((d_h, d_lat), lambda i: (0, 0)),
            pl.BlockSpec((1, d_h), lambda i: (0, 0)),
        ],
        out_specs=pl.BlockSpec((br, d_lat), lambda i: (i, 0)),
        out_shape=jax.ShapeDtypeStruct((n, d_lat), _f32),
    )(p1, pdeg, W2, b1.reshape(1, d_h))

    p2 = _make_seg_sum(n, npad, d_lat, n_chunks_per_tile, 50)(h2p, src2d, dst2d)
    p2 = p2.reshape(NC, npad, d_lat)

    bi = 400
    adj, z = pl.pallas_call(
        functools.partial(_dec_body, n=n, bi=bi),
        grid=(n // bi,),
        in_specs=[
            pl.BlockSpec((NC, npad, d_lat), lambda i: (0, 0, 0)),
            pl.BlockSpec((NC, npad, 16), lambda i: (0, 0, 0)),
            pl.BlockSpec((1, d_lat), lambda i: (0, 0)),
        ],
        out_specs=[
            pl.BlockSpec((bi, n), lambda i: (i, 0)),
            pl.BlockSpec((n, d_lat), lambda i: (0, 0)),
        ],
        out_shape=[
            jax.ShapeDtypeStruct((n, n), _f32),
            jax.ShapeDtypeStruct((n, d_lat), _f32),
        ],
    )(p2, pdeg, b2.reshape(1, d_lat))

    return (adj, z, z, z)


# speedup vs baseline: 1.0289x; 1.0289x over previous
import functools

import jax
import jax.numpy as jnp
from jax import lax
from jax.experimental import pallas as pl
from jax.experimental.pallas import tpu as pltpu
from jax.experimental.pallas import tpu_sc as plsc

NC = 2
NS = 16
NW = NC * NS
CHUNK = 100

_f32 = jnp.float32


def _mesh():
    return plsc.VectorSubcoreMesh(
        core_axis_name="c", subcore_axis_name="s", num_cores=NC, num_subcores=NS
    )


def _fill_zeros(ref, n_rows, d):
    zeros16 = jnp.zeros((16,), _f32)

    def body(i, _):
        for k in range(d // 16):
            ref[i, pl.ds(k * 16, 16)] = zeros16
        return 0

    lax.fori_loop(0, n_rows, body, 0)


def _make_seg_sum(n, npad, d, n_chunks_per_tile, k):
    rows_per_tile = npad // NS
    assert n_chunks_per_tile % k == 0
    assert 2 * k * CHUNK >= rows_per_tile
    nb = n_chunks_per_tile // k

    @functools.partial(
        pl.kernel,
        out_type=jax.ShapeDtypeStruct((NC * npad, d), _f32),
        mesh=_mesh(),
        compiler_params=pltpu.CompilerParams(use_tc_tiling_on_sc=False),
        scratch_types=[
            pltpu.VMEM((n_chunks_per_tile, CHUNK), jnp.int32),
            pltpu.VMEM((n_chunks_per_tile, CHUNK), jnp.int32),
            pltpu.VMEM((2 * k * CHUNK, d), _f32),
            pltpu.VMEM_SHARED((npad, d), _f32),
            pltpu.SemaphoreType.DMA,
            pltpu.SemaphoreType.DMA,
        ],
    )
    def seg_sum(h_hbm, src_hbm, dst_hbm, out_hbm, sidx, didx, rows, acc, semg, sems):
        cid = lax.axis_index("c")
        sid = lax.axis_index("s")
        wid = cid * NS + sid

        _fill_zeros(rows, rows_per_tile, d)
        pltpu.sync_copy(
            rows.at[pl.ds(0, rows_per_tile)],
            acc.at[pl.ds(sid * rows_per_tile, rows_per_tile)],
        )

        pltpu.sync_copy(src_hbm.at[wid], sidx)
        pltpu.sync_copy(dst_hbm.at[wid], didx)
        plsc.subcore_barrier()

        gs = k * CHUNK

        def fire_g(b, grp):
            def f(j2, _):
                pltpu.async_copy(
                    h_hbm.at[sidx.at[b * k + j2]],
                    rows.at[pl.ds(grp * gs + j2 * CHUNK, CHUNK)],
                    semg,
                )
                return 0
            lax.fori_loop(0, k, f, 0)

        def drain_g(b, grp):
            def f(j2, _):
                pltpu.make_async_copy(
                    h_hbm.at[sidx.at[b * k + j2]],
                    rows.at[pl.ds(grp * gs + j2 * CHUNK, CHUNK)],
                    semg,
                ).wait()
                return 0
            lax.fori_loop(0, k, f, 0)

        def fire_s(b, grp):
            def f(j2, _):
                pltpu.async_copy(
                    rows.at[pl.ds(grp * gs + j2 * CHUNK, CHUNK)],
                    acc.at[didx.at[b * k + j2]],
                    sems,
                    add=True,
                )
                return 0
            lax.fori_loop(0, k, f, 0)

        def drain_s(b, grp):
            def f(j2, _):
                pltpu.make_async_copy(
                    rows.at[pl.ds(grp * gs + j2 * CHUNK, CHUNK)],
                    acc.at[didx.at[b * k + j2]],
                    sems,
                ).wait()
                return 0
            lax.fori_loop(0, k, f, 0)

        fire_g(0, 0)

        def pipe(b, _):
            grp = lax.rem(b, 2)
            drain_g(b, grp)

            @pl.when(b + 1 < nb)
            def _():
                fire_g(b + 1, 1 - grp)

            fire_s(b, grp)
            drain_s(b, grp)
            return 0

        lax.fori_loop(0, nb, pipe, 0)
        plsc.subcore_barrier()

        r0 = sid * rows_per_tile
        pltpu.sync_copy(
            acc.at[pl.ds(r0, rows_per_tile)],
            out_hbm.at[pl.ds(cid * npad + r0, rows_per_tile)],
        )

    return seg_sum


def _make_deg(n, npad, n_chunks_per_tile):
    rows_per_tile = npad // NS

    @functools.partial(
        pl.kernel,
        out_type=jax.ShapeDtypeStruct((NC * npad, 16), _f32),
        mesh=_mesh(),
        compiler_params=pltpu.CompilerParams(use_tc_tiling_on_sc=False),
        scratch_types=[
            pltpu.VMEM((n_chunks_per_tile, CHUNK), jnp.int32),
            pltpu.VMEM((CHUNK, 16), _f32),
            pltpu.VMEM((rows_per_tile, 16), _f32),
            pltpu.VMEM_SHARED((npad, 16), _f32),
            pltpu.SemaphoreType.DMA,
        ],
    )
    def deg_kernel(dst_hbm, out_hbm, didx, ones_v, zb, acc, sem):
        cid = lax.axis_index("c")
        sid = lax.axis_index("s")
        wid = cid * NS + sid

        ones16 = jnp.ones((16,), _f32)

        def fill_ones(i, _):
            ones_v[i, :] = ones16
            return 0

        lax.fori_loop(0, CHUNK, fill_ones, 0)
        _fill_zeros(zb, rows_per_tile, 16)
        pltpu.sync_copy(zb, acc.at[pl.ds(sid * rows_per_tile, rows_per_tile)])

        pltpu.sync_copy(dst_hbm.at[wid], didx)
        plsc.subcore_barrier()

        def fire(j, _):
            pltpu.async_copy(ones_v, acc.at[didx.at[j]], sem, add=True)
            return 0

        def drain(j, _):
            pltpu.make_async_copy(ones_v, acc.at[didx.at[j]], sem).wait()
            return 0

        lax.fori_loop(0, n_chunks_per_tile, fire, 0)
        lax.fori_loop(0, n_chunks_per_tile, drain, 0)
        plsc.subcore_barrier()

        r0 = sid * rows_per_tile
        pltpu.sync_copy(
            acc.at[pl.ds(r0, rows_per_tile)],
            out_hbm.at[pl.ds(cid * npad + r0, rows_per_tile)],
        )

    return deg_kernel


def _dinv_from_parts(p0, p1):
    deg = p0[:, 0:1] + p1[:, 0:1]
    return lax.rsqrt(jnp.maximum(deg, 1.0))


def _mm_body(x_ref, w1_ref, o_ref):
    o_ref[...] = jnp.dot(x_ref[...], w1_ref[...], preferred_element_type=_f32)


def _prescale_body(h_ref, pd_ref, o_ref):
    dinv = _dinv_from_parts(pd_ref[0], pd_ref[1])
    o_ref[...] = h_ref[...] * dinv


def _h2_body(p1_ref, pd_ref, w2_ref, b1_ref, o_ref):
    dinv = _dinv_from_parts(pd_ref[0], pd_ref[1])
    s1 = p1_ref[0] + p1_ref[1]
    hidden = jax.nn.relu(dinv * s1 + b1_ref[...])
    o_ref[...] = jnp.dot(hidden, w2_ref[...], preferred_element_type=_f32) * dinv


def _dec_body(p2_ref, pd_ref, b2_ref, o_ref, z_ref, *, n, bi):
    i = pl.program_id(0)

    @pl.when(i == 0)
    def _():
        dinv = _dinv_from_parts(pd_ref[0, :n], pd_ref[1, :n])
        z_ref[...] = dinv * (p2_ref[0, :n] + p2_ref[1, :n]) + b2_ref[...]

    zi = z_ref[pl.ds(i * bi, bi), :]
    prod = lax.dot_general(
        zi, z_ref[...], (((1,), (1,)), ((), ())),
        preferred_element_type=_f32,
    )
    o_ref[...] = 0.5 * jnp.tanh(0.5 * prod) + 0.5


def kernel(X, edge_index, W1, b1, W2, b2):
    n, d_in = X.shape
    d_h = W1.shape[1]
    d_lat = W2.shape[1]
    e = edge_index.shape[1]

    n_chunks_per_tile = e // (NW * CHUNK)
    ei = edge_index.reshape(2, NW, n_chunks_per_tile, CHUNK)
    src2d, dst2d = ei[0], ei[1]
    npad = ((n + NS * 8 - 1) // (NS * 8)) * NS * 8

    pdeg = _make_deg(n, npad, n_chunks_per_tile)(dst2d).reshape(NC, npad, 16)

    br = 1000
    grid = (n // br,)
    h1 = pl.pallas_call(
        _mm_body,
        grid=grid,
        in_specs=[
            pl.BlockSpec((br, d_in), lambda i: (i, 0)),
            pl.BlockSpec((d_in, d_h), lambda i: (0, 0)),
        ],
        out_specs=pl.BlockSpec((br, d_h), lambda i: (i, 0)),
        out_shape=jax.ShapeDtypeStruct((n, d_h), _f32),
    )(X, W1)

    h1p = pl.pallas_call(
        _prescale_body,
        grid=grid,
        in_specs=[
            pl.BlockSpec((br, d_h), lambda i: (i, 0)),
            pl.BlockSpec((NC, br, 16), lambda i: (0, i, 0)),
        ],
        out_specs=pl.BlockSpec((br, d_h), lambda i: (i, 0)),
        out_shape=jax.ShapeDtypeStruct((n, d_h), _f32),
    )(h1, pdeg)

    p1 = _make_seg_sum(n, npad, d_h, n_chunks_per_tile, 5)(h1p, src2d, dst2d)
    p1 = p1.reshape(NC, npad, d_h)

    h2p = pl.pallas_call(
        _h2_body,
        grid=grid,
        in_specs=[
            pl.BlockSpec((NC, br, d_h), lambda i: (0, i, 0)),
            pl.BlockSpec((NC, br, 16), lambda i: (0, i, 0)),
            pl.BlockSpec((d_h, d_lat), lambda i: (0, 0)),
            pl.BlockSpec((1, d_h), lambda i: (0, 0)),
        ],
        out_specs=pl.BlockSpec((br, d_lat), lambda i: (i, 0)),
        out_shape=jax.ShapeDtypeStruct((n, d_lat), _f32),
    )(p1, pdeg, W2, b1.reshape(1, d_h))

    p2 = _make_seg_sum(n, npad, d_lat, n_chunks_per_tile, 25)(h2p, src2d, dst2d)
    p2 = p2.reshape(NC, npad, d_lat)

    bi = 400
    adj, z = pl.pallas_call(
        functools.partial(_dec_body, n=n, bi=bi),
        grid=(n // bi,),
        in_specs=[
            pl.BlockSpec((NC, npad, d_lat), lambda i: (0, 0, 0)),
            pl.BlockSpec((NC, npad, 16), lambda i: (0, 0, 0)),
            pl.BlockSpec((1, d_lat), lambda i: (0, 0)),
        ],
        out_specs=[
            pl.BlockSpec((bi, n), lambda i: (i, 0)),
            pl.BlockSpec((n, d_lat), lambda i: (0, 0)),
        ],
        out_shape=[
            jax.ShapeDtypeStruct((n, n), _f32),
            jax.ShapeDtypeStruct((n, d_lat), _f32),
        ],
    )(p2, pdeg, b2.reshape(1, d_lat))

    return (adj, z, z, z)
